# baseline (speedup 1.0000x reference)
import functools

import jax
import jax.numpy as jnp
from jax import lax
from jax.experimental import pallas as pl
from jax.experimental.pallas import tpu as pltpu
from jax.experimental.pallas import tpu_sc as plsc

D = 128
NC, NS = 2, 16
NW = NC * NS
N = 100000
CHUNK = 224
NCHUNKS = 14
PER_W = CHUNK * NCHUNKS
IDXW = 7 * CHUNK
OUTW = CHUNK * D

B2 = 124 * D
B3 = B2 + 273 * D
B4 = B3 + 224 * D
TAB_ELEMS = B4 + 63 * D


def _make_sc_call():
    mesh = plsc.VectorSubcoreMesh(
        core_axis_name="c", subcore_axis_name="s", num_cores=NC, num_subcores=NS
    )

    @functools.partial(
        pl.kernel,
        out_type=jax.ShapeDtypeStruct((N * D,), jnp.float32),
        mesh=mesh,
        compiler_params=pltpu.CompilerParams(needs_layout_passes=False),
        scratch_types=[
            pltpu.VMEM((TAB_ELEMS // 2,), jnp.int32),
            pltpu.VMEM((2 * IDXW,), jnp.int32),
            pltpu.VMEM((2 * OUTW,), jnp.float32),
            pltpu.SemaphoreType.DMA,
            pltpu.SemaphoreType.DMA,
        ],
    )
    def sc_embed(tab_h, an_h, fc_h, dg_h, ct_h, nh_h, ar_h, hy_h, out_h,
                 tab_v, idx_v, out_v, idx_sem, out_sem):
        wid = lax.axis_index("s") * NC + lax.axis_index("c")
        idx_refs = (an_h, fc_h, dg_h, ct_h, nh_h, ar_h, hy_h)
        pltpu.sync_copy(tab_h, tab_v)
        base = jnp.minimum(wid * PER_W, N - PER_W)

        def issue_idx(ci, p):
            nb = base + ci * CHUNK
            for f, r in enumerate(idx_refs):
                pltpu.async_copy(r.at[pl.ds(nb, CHUNK)],
                                 idx_v.at[pl.ds(p * IDXW + f * CHUNK, CHUNK)],
                                 idx_sem)

        def drain_idx(p):
            pltpu.make_async_copy(an_h.at[pl.ds(0, IDXW)],
                                  idx_v.at[pl.ds(p * IDXW, IDXW)],
                                  idx_sem).wait()

        def drain_out(ci, p):
            nb = base + ci * CHUNK
            pltpu.make_async_copy(out_v.at[pl.ds(p * OUTW, OUTW)],
                                  out_h.at[pl.ds(nb * D, OUTW)],
                                  out_sem).wait()

        issue_idx(0, 0)

        def chunk_body(ci, carry):
            p = lax.rem(ci, 2)
            nb = base + ci * CHUNK
            pofs = p * IDXW
            oofs = p * OUTW

            @pl.when(ci >= 2)
            def _():
                drain_out(ci - 2, p)

            drain_idx(p)

            @pl.when(ci < NCHUNKS - 1)
            def _():
                issue_idx(ci + 1, 1 - p)

            @plsc.parallel_loop(0, CHUNK // 16, 1, unroll=4)
            def blk_body(b):
                o = b * 16

                def ld(f):
                    return idx_v[pl.ds(pofs + f * CHUNK + o, 16)]

                an, fc, dg, ct, nh, ar, hy = (ld(f) for f in range(7))
                dw = D // 2
                c1 = an * dw
                c2 = (fc * 13 + hy) * dw + B2 // 2
                c3 = (dg * 14 + nh) * dw + B3 // 2
                c4 = (ct * 7 + ar) * dw + B4 // 2

                mask_hi = jnp.full((16,), -65536, jnp.int32)

                for n in range(16):
                    rb1, rb2, rb3, rb4 = c1[n], c2[n], c3[n], c4[n]
                    oo = oofs + (o + n) * D
                    for j in range(4):
                        a0 = plsc.bitcast(tab_v[pl.ds(rb1 + j * 16, 16)], jnp.bfloat16)
                        a1 = plsc.bitcast(tab_v[pl.ds(rb2 + j * 16, 16)], jnp.bfloat16)
                        a2 = plsc.bitcast(tab_v[pl.ds(rb3 + j * 16, 16)], jnp.bfloat16)
                        a3 = plsc.bitcast(tab_v[pl.ds(rb4 + j * 16, 16)], jnp.bfloat16)
                        s = plsc.bitcast((a0 + a1) + (a2 + a3), jnp.int32)
                        lo = plsc.bitcast(lax.shift_left(s, 16), jnp.float32)
                        hi = plsc.bitcast(lax.bitwise_and(s, mask_hi), jnp.float32)
                        out_v[pl.ds(oo + j * 32, 16)] = lo
                        out_v[pl.ds(oo + j * 32 + 16, 16)] = hi

            pltpu.async_copy(out_v.at[pl.ds(oofs, OUTW)],
                             out_h.at[pl.ds(nb * D, OUTW)],
                             out_sem)
            return carry

        lax.fori_loop(0, NCHUNKS, chunk_body, 0)
        drain_out(NCHUNKS - 2, 0)
        drain_out(NCHUNKS - 1, 1)

    return sc_embed


def kernel(atomic_num, formal_charge, degree, chiral_tag, total_numHs,
           is_aromatic, hybridization, W_atomic_num, W_formal_charge,
           W_degree, W_chiral_tag, W_total_numHs, W_is_aromatic,
           W_hybridization):
    idxs = [
        x.astype(jnp.int32)
        for x in (atomic_num, formal_charge, degree, chiral_tag,
                  total_numHs, is_aromatic, hybridization)
    ]
    w2 = (W_formal_charge[:, None, :] + W_hybridization[None, :, :]).reshape(-1, D)
    w3 = (W_degree[:, None, :] + W_total_numHs[None, :, :]).reshape(-1, D)
    w4 = (W_chiral_tag[:, None, :] + W_is_aromatic[None, :, :]).reshape(-1, D)
    tab = jnp.concatenate([W_atomic_num, w2, w3, w4], axis=0)
    tab = (tab.reshape(-1, 4, 2, 16).transpose(0, 1, 3, 2)
           .astype(jnp.bfloat16).reshape(-1, 2))
    tab = lax.bitcast_convert_type(tab, jnp.int32).reshape(-1)

    out_flat = _make_sc_call()(tab, *idxs)
    return out_flat.reshape(N, D)


# speedup vs baseline: 1.5437x; 1.5437x over previous
import functools

import jax
import jax.numpy as jnp
from jax import lax
from jax.experimental import pallas as pl
from jax.experimental.pallas import tpu as pltpu
from jax.experimental.pallas import tpu_sc as plsc

D = 128
NC, NS = 2, 16
NW = NC * NS
N = 100000
CHUNK = 224
NCHUNKS = 14
PER_W = CHUNK * NCHUNKS
IDXW = 7 * CHUNK
OUTW = CHUNK * D

B2 = 124 * D
B3 = B2 + 273 * D
B4 = B3 + 224 * D
TAB_ELEMS = B4 + 63 * D


def _make_sc_call():
    mesh = plsc.VectorSubcoreMesh(
        core_axis_name="c", subcore_axis_name="s", num_cores=NC, num_subcores=NS
    )

    @functools.partial(
        pl.kernel,
        out_type=jax.ShapeDtypeStruct((N * D,), jnp.float32),
        mesh=mesh,
        compiler_params=pltpu.CompilerParams(needs_layout_passes=False),
        scratch_types=[
            pltpu.VMEM((TAB_ELEMS // 2,), jnp.int32),
            pltpu.VMEM((2 * IDXW,), jnp.int32),
            pltpu.VMEM((2 * OUTW,), jnp.float32),
            pltpu.SemaphoreType.DMA,
            pltpu.SemaphoreType.DMA,
        ],
    )
    def sc_embed(tab_h, an_h, fc_h, dg_h, ct_h, nh_h, ar_h, hy_h, out_h,
                 tab_v, idx_v, out_v, idx_sem, out_sem):
        wid = lax.axis_index("s") * NC + lax.axis_index("c")
        idx_refs = (an_h, fc_h, dg_h, ct_h, nh_h, ar_h, hy_h)
        pltpu.sync_copy(tab_h, tab_v)
        base = jnp.minimum(wid * PER_W, N - PER_W)

        def issue_idx(ci, p):
            nb = base + ci * CHUNK
            for f, r in enumerate(idx_refs):
                pltpu.async_copy(r.at[pl.ds(nb, CHUNK)],
                                 idx_v.at[pl.ds(p * IDXW + f * CHUNK, CHUNK)],
                                 idx_sem)

        def drain_idx(p):
            pltpu.make_async_copy(an_h.at[pl.ds(0, IDXW)],
                                  idx_v.at[pl.ds(p * IDXW, IDXW)],
                                  idx_sem).wait()

        def drain_out(ci, p):
            nb = base + ci * CHUNK
            pltpu.make_async_copy(out_v.at[pl.ds(p * OUTW, OUTW)],
                                  out_h.at[pl.ds(nb * D, OUTW)],
                                  out_sem).wait()

        issue_idx(0, 0)

        def chunk_body(ci, carry):
            p = lax.rem(ci, 2)
            nb = base + ci * CHUNK
            pofs = p * IDXW
            oofs = p * OUTW

            @pl.when(ci >= 2)
            def _():
                drain_out(ci - 2, p)

            drain_idx(p)

            @pl.when(ci < NCHUNKS - 1)
            def _():
                issue_idx(ci + 1, 1 - p)

            @plsc.parallel_loop(0, CHUNK // 16, 1, unroll=2)
            def blk_body(b):
                o = b * 16

                def ld(f):
                    return idx_v[pl.ds(pofs + f * CHUNK + o, 16)]

                an, fc, dg, ct, nh, ar, hy = (ld(f) for f in range(7))
                dw = D // 2
                c1 = an * dw
                c2 = (fc * 13 + hy) * dw + B2 // 2
                c3 = (dg * 14 + nh) * dw + B3 // 2
                c4 = (ct * 7 + ar) * dw + B4 // 2

                mask_hi = jnp.full((16,), -65536, jnp.int32)

                rbs = [(c1[n], c2[n], c3[n], c4[n]) for n in range(16)]
                for j in range(4):
                    for n in range(16):
                        rb1, rb2, rb3, rb4 = rbs[n]
                        oo = oofs + (o + n) * D
                        a0 = plsc.bitcast(tab_v[pl.ds(rb1 + j * 16, 16)], jnp.bfloat16)
                        a1 = plsc.bitcast(tab_v[pl.ds(rb2 + j * 16, 16)], jnp.bfloat16)
                        a2 = plsc.bitcast(tab_v[pl.ds(rb3 + j * 16, 16)], jnp.bfloat16)
                        a3 = plsc.bitcast(tab_v[pl.ds(rb4 + j * 16, 16)], jnp.bfloat16)
                        s = plsc.bitcast((a0 + a1) + (a2 + a3), jnp.int32)
                        lo = plsc.bitcast(lax.shift_left(s, 16), jnp.float32)
                        hi = plsc.bitcast(lax.bitwise_and(s, mask_hi), jnp.float32)
                        out_v[pl.ds(oo + j * 32, 16)] = lo
                        out_v[pl.ds(oo + j * 32 + 16, 16)] = hi

            pltpu.async_copy(out_v.at[pl.ds(oofs, OUTW)],
                             out_h.at[pl.ds(nb * D, OUTW)],
                             out_sem)
            return carry

        lax.fori_loop(0, NCHUNKS, chunk_body, 0)
        drain_out(NCHUNKS - 2, 0)
        drain_out(NCHUNKS - 1, 1)

    return sc_embed


def kernel(atomic_num, formal_charge, degree, chiral_tag, total_numHs,
           is_aromatic, hybridization, W_atomic_num, W_formal_charge,
           W_degree, W_chiral_tag, W_total_numHs, W_is_aromatic,
           W_hybridization):
    idxs = [
        x.astype(jnp.int32)
        for x in (atomic_num, formal_charge, degree, chiral_tag,
                  total_numHs, is_aromatic, hybridization)
    ]
    w2 = (W_formal_charge[:, None, :] + W_hybridization[None, :, :]).reshape(-1, D)
    w3 = (W_degree[:, None, :] + W_total_numHs[None, :, :]).reshape(-1, D)
    w4 = (W_chiral_tag[:, None, :] + W_is_aromatic[None, :, :]).reshape(-1, D)
    tab = jnp.concatenate([W_atomic_num, w2, w3, w4], axis=0)
    tab = (tab.reshape(-1, 4, 2, 16).transpose(0, 1, 3, 2)
           .astype(jnp.bfloat16).reshape(-1, 2))
    tab = lax.bitcast_convert_type(tab, jnp.int32).reshape(-1)

    out_flat = _make_sc_call()(tab, *idxs)
    return out_flat.reshape(N, D)


# speedup vs baseline: 1.6543x; 1.0717x over previous
import functools

import jax
import jax.numpy as jnp
from jax import lax
from jax.experimental import pallas as pl
from jax.experimental.pallas import tpu as pltpu
from jax.experimental.pallas import tpu_sc as plsc

D = 128
NC, NS = 2, 16
NW = NC * NS
N = 100000
CHUNK = 224
NCHUNKS = 14
PER_W = CHUNK * NCHUNKS
IDXW = 7 * CHUNK
OUTW = CHUNK * D

B2 = 124 * D
B3 = B2 + 273 * D
B4 = B3 + 224 * D
TAB_ELEMS = B4 + 63 * D


def _make_sc_call():
    mesh = plsc.VectorSubcoreMesh(
        core_axis_name="c", subcore_axis_name="s", num_cores=NC, num_subcores=NS
    )

    @functools.partial(
        pl.kernel,
        out_type=jax.ShapeDtypeStruct((N * D,), jnp.float32),
        mesh=mesh,
        compiler_params=pltpu.CompilerParams(needs_layout_passes=False),
        scratch_types=[
            pltpu.VMEM((TAB_ELEMS // 2,), jnp.int32),
            pltpu.VMEM((2 * IDXW,), jnp.int32),
            pltpu.VMEM((2 * OUTW,), jnp.float32),
            pltpu.SemaphoreType.DMA,
            pltpu.SemaphoreType.DMA,
        ],
    )
    def sc_embed(tab_h, an_h, fc_h, dg_h, ct_h, nh_h, ar_h, hy_h, out_h,
                 tab_v, idx_v, out_v, idx_sem, out_sem):
        wid = lax.axis_index("s") * NC + lax.axis_index("c")
        idx_refs = (an_h, fc_h, dg_h, ct_h, nh_h, ar_h, hy_h)
        base = jnp.minimum(wid * PER_W, N - PER_W)

        def issue_idx(ci, p):
            nb = base + ci * CHUNK
            for f, r in enumerate(idx_refs):
                pltpu.async_copy(r.at[pl.ds(nb, CHUNK)],
                                 idx_v.at[pl.ds(p * IDXW + f * CHUNK, CHUNK)],
                                 idx_sem)

        def drain_idx(p):
            pltpu.make_async_copy(an_h.at[pl.ds(0, IDXW)],
                                  idx_v.at[pl.ds(p * IDXW, IDXW)],
                                  idx_sem).wait()

        def drain_out(ci, p):
            nb = base + ci * CHUNK
            pltpu.make_async_copy(out_v.at[pl.ds(p * OUTW, OUTW)],
                                  out_h.at[pl.ds(nb * D, OUTW)],
                                  out_sem).wait()

        issue_idx(0, 0)
        pltpu.async_copy(tab_h, tab_v, out_sem).wait()

        def chunk_body(ci, carry):
            p = lax.rem(ci, 2)
            nb = base + ci * CHUNK
            pofs = p * IDXW
            oofs = p * OUTW

            @pl.when(ci >= 2)
            def _():
                drain_out(ci - 2, p)

            drain_idx(p)

            @pl.when(ci < NCHUNKS - 1)
            def _():
                issue_idx(ci + 1, 1 - p)

            @plsc.parallel_loop(0, CHUNK // 16, 1, unroll=2)
            def blk_body(b):
                o = b * 16

                def ld(f):
                    return idx_v[pl.ds(pofs + f * CHUNK + o, 16)]

                an, fc, dg, ct, nh, ar, hy = (ld(f) for f in range(7))
                dw = D // 2
                c1 = an * dw
                c2 = (fc * 13 + hy) * dw + B2 // 2
                c3 = (dg * 14 + nh) * dw + B3 // 2
                c4 = (ct * 7 + ar) * dw + B4 // 2

                mask_hi = jnp.full((16,), -65536, jnp.int32)

                for n in range(16):
                    rb1, rb2, rb3, rb4 = c1[n], c2[n], c3[n], c4[n]
                    oo = oofs + (o + n) * D
                    for j in range(4):
                        a0 = plsc.bitcast(tab_v[pl.ds(rb1 + j * 16, 16)], jnp.bfloat16)
                        a1 = plsc.bitcast(tab_v[pl.ds(rb2 + j * 16, 16)], jnp.bfloat16)
                        a2 = plsc.bitcast(tab_v[pl.ds(rb3 + j * 16, 16)], jnp.bfloat16)
                        a3 = plsc.bitcast(tab_v[pl.ds(rb4 + j * 16, 16)], jnp.bfloat16)
                        s = plsc.bitcast((a0 + a1) + (a2 + a3), jnp.int32)
                        lo = plsc.bitcast(lax.shift_left(s, 16), jnp.float32)
                        hi = plsc.bitcast(lax.bitwise_and(s, mask_hi), jnp.float32)
                        out_v[pl.ds(oo + j * 32, 16)] = lo
                        out_v[pl.ds(oo + j * 32 + 16, 16)] = hi

            pltpu.async_copy(out_v.at[pl.ds(oofs, OUTW)],
                             out_h.at[pl.ds(nb * D, OUTW)],
                             out_sem)
            return carry

        lax.fori_loop(0, NCHUNKS, chunk_body, 0)
        drain_out(NCHUNKS - 2, 0)
        drain_out(NCHUNKS - 1, 1)

    return sc_embed


def kernel(atomic_num, formal_charge, degree, chiral_tag, total_numHs,
           is_aromatic, hybridization, W_atomic_num, W_formal_charge,
           W_degree, W_chiral_tag, W_total_numHs, W_is_aromatic,
           W_hybridization):
    idxs = [
        x.astype(jnp.int32)
        for x in (atomic_num, formal_charge, degree, chiral_tag,
                  total_numHs, is_aromatic, hybridization)
    ]
    w2 = (W_formal_charge[:, None, :] + W_hybridization[None, :, :]).reshape(-1, D)
    w3 = (W_degree[:, None, :] + W_total_numHs[None, :, :]).reshape(-1, D)
    w4 = (W_chiral_tag[:, None, :] + W_is_aromatic[None, :, :]).reshape(-1, D)
    tab = jnp.concatenate([W_atomic_num, w2, w3, w4], axis=0)
    tab = (tab.reshape(-1, 4, 2, 16).transpose(0, 1, 3, 2)
           .astype(jnp.bfloat16).reshape(-1, 2))
    tab = lax.bitcast_convert_type(tab, jnp.int32).reshape(-1)

    out_flat = _make_sc_call()(tab, *idxs)
    return out_flat.reshape(N, D)


# speedup vs baseline: 2.7284x; 1.6492x over previous
import functools

import jax
import jax.numpy as jnp
from jax import lax
from jax.experimental import pallas as pl
from jax.experimental.pallas import tpu as pltpu
from jax.experimental.pallas import tpu_sc as plsc

D = 128
NC, NS = 2, 16
NW = NC * NS
N = 100000
CHUNK = 224
NCHUNKS = 14
PER_W = CHUNK * NCHUNKS
IDXW = 7 * CHUNK
OUTW = CHUNK * D

B2 = 124 * D
B3 = B2 + 273 * D
B4 = B3 + 224 * D
TAB_ELEMS = B4 + 63 * D


def _make_sc_call():
    mesh = plsc.VectorSubcoreMesh(
        core_axis_name="c", subcore_axis_name="s", num_cores=NC, num_subcores=NS
    )

    @functools.partial(
        pl.kernel,
        out_type=jax.ShapeDtypeStruct((N * D,), jnp.float32),
        mesh=mesh,
        compiler_params=pltpu.CompilerParams(needs_layout_passes=False),
        scratch_types=[
            pltpu.VMEM((TAB_ELEMS // 2,), jnp.int32),
            pltpu.VMEM((2 * IDXW,), jnp.int32),
            pltpu.VMEM((2 * OUTW,), jnp.float32),
            pltpu.SemaphoreType.DMA,
            pltpu.SemaphoreType.DMA,
        ],
    )
    def sc_embed(tab_h, an_h, fc_h, dg_h, ct_h, nh_h, ar_h, hy_h, out_h,
                 tab_v, idx_v, out_v, idx_sem, out_sem):
        wid = lax.axis_index("s") * NC + lax.axis_index("c")
        idx_refs = (an_h, fc_h, dg_h, ct_h, nh_h, ar_h, hy_h)
        base = jnp.minimum(wid * PER_W, N - PER_W)

        def issue_idx(ci, p):
            nb = base + ci * CHUNK
            for f, r in enumerate(idx_refs):
                pltpu.async_copy(r.at[pl.ds(nb, CHUNK)],
                                 idx_v.at[pl.ds(p * IDXW + f * CHUNK, CHUNK)],
                                 idx_sem)

        def drain_idx(p):
            pltpu.make_async_copy(an_h.at[pl.ds(0, IDXW)],
                                  idx_v.at[pl.ds(p * IDXW, IDXW)],
                                  idx_sem).wait()

        def drain_out(ci, p):
            nb = base + ci * CHUNK
            pltpu.make_async_copy(out_v.at[pl.ds(p * OUTW, OUTW)],
                                  out_h.at[pl.ds(nb * D, OUTW)],
                                  out_sem).wait()

        issue_idx(0, 0)
        pltpu.async_copy(tab_h, tab_v, out_sem).wait()

        def chunk_body(ci, carry):
            p = lax.rem(ci, 2)
            nb = base + ci * CHUNK
            pofs = p * IDXW
            oofs = p * OUTW

            @pl.when(ci >= 2)
            def _():
                drain_out(ci - 2, p)

            drain_idx(p)

            @pl.when(ci < NCHUNKS - 1)
            def _():
                issue_idx(ci + 1, 1 - p)

            @plsc.parallel_loop(0, CHUNK // 16, 1, unroll=1)
            def blk_body(b):
                o = b * 16

                def ld(f):
                    return idx_v[pl.ds(pofs + f * CHUNK + o, 16)]

                an, fc, dg, ct, nh, ar, hy = (ld(f) for f in range(7))
                dw = D // 2
                c1 = an * dw
                c2 = (fc * 13 + hy) * dw + B2 // 2
                c3 = (dg * 14 + nh) * dw + B3 // 2
                c4 = (ct * 7 + ar) * dw + B4 // 2

                mask_hi = jnp.full((16,), -65536, jnp.int32)

                for n in range(16):
                    rb1, rb2, rb3, rb4 = c1[n], c2[n], c3[n], c4[n]
                    oo = oofs + (o + n) * D
                    for j in range(4):
                        a0 = plsc.bitcast(tab_v[pl.ds(rb1 + j * 16, 16)], jnp.bfloat16)
                        a1 = plsc.bitcast(tab_v[pl.ds(rb2 + j * 16, 16)], jnp.bfloat16)
                        a2 = plsc.bitcast(tab_v[pl.ds(rb3 + j * 16, 16)], jnp.bfloat16)
                        a3 = plsc.bitcast(tab_v[pl.ds(rb4 + j * 16, 16)], jnp.bfloat16)
                        s = plsc.bitcast((a0 + a1) + (a2 + a3), jnp.int32)
                        lo = plsc.bitcast(lax.shift_left(s, 16), jnp.float32)
                        hi = plsc.bitcast(lax.bitwise_and(s, mask_hi), jnp.float32)
                        out_v[pl.ds(oo + j * 32, 16)] = lo
                        out_v[pl.ds(oo + j * 32 + 16, 16)] = hi

            pltpu.async_copy(out_v.at[pl.ds(oofs, OUTW)],
                             out_h.at[pl.ds(nb * D, OUTW)],
                             out_sem)
            return carry

        lax.fori_loop(0, NCHUNKS, chunk_body, 0)
        drain_out(NCHUNKS - 2, 0)
        drain_out(NCHUNKS - 1, 1)

    return sc_embed


def kernel(atomic_num, formal_charge, degree, chiral_tag, total_numHs,
           is_aromatic, hybridization, W_atomic_num, W_formal_charge,
           W_degree, W_chiral_tag, W_total_numHs, W_is_aromatic,
           W_hybridization):
    idxs = [
        x.astype(jnp.int32)
        for x in (atomic_num, formal_charge, degree, chiral_tag,
                  total_numHs, is_aromatic, hybridization)
    ]
    w2 = (W_formal_charge[:, None, :] + W_hybridization[None, :, :]).reshape(-1, D)
    w3 = (W_degree[:, None, :] + W_total_numHs[None, :, :]).reshape(-1, D)
    w4 = (W_chiral_tag[:, None, :] + W_is_aromatic[None, :, :]).reshape(-1, D)
    tab = jnp.concatenate([W_atomic_num, w2, w3, w4], axis=0)
    tab = (tab.reshape(-1, 4, 2, 16).transpose(0, 1, 3, 2)
           .astype(jnp.bfloat16).reshape(-1, 2))
    tab = lax.bitcast_convert_type(tab, jnp.int32).reshape(-1)

    out_flat = _make_sc_call()(tab, *idxs)
    return out_flat.reshape(N, D)
